# baseline (speedup 1.0000x reference)
import jax
import jax.numpy as jnp
from jax import lax
from jax.experimental import pallas as pl
from jax.experimental.pallas import tpu as pltpu
from jax.experimental.pallas import tpu_sc as plsc

N = 100000
DA = 128
L = 16
NC = 2
NS = 16
NW = NC * NS
R = 128
NFULL = N // R
TAIL = N - NFULL * R
TAIL_BASE = NFULL * R
K = DA // L

BT = 1000
NT = 64000
C0 = NT // R
NSC = NFULL - C0


def _body(x_hbm, v_hbm, u_hbm, b_hbm, beff_hbm, eff_hbm, y_hbm,
          xin0, xin1, yout0, yout1, effb0, effb1,
          vvm, uvm, bvm, beffvm, isem0, isem1, osem0, osem1):
    wid = lax.axis_index("s") * NC + lax.axis_index("c")

    pltpu.sync_copy(v_hbm, vvm)
    pltpu.sync_copy(u_hbm, uvm)
    pltpu.sync_copy(b_hbm, bvm)
    pltpu.sync_copy(beff_hbm, beffvm)

    vv = [vvm[pl.ds(k * L, L)] for k in range(K)]
    uu = [uvm[pl.ds(k * L, L)] for k in range(K)]
    bb = [bvm[pl.ds(k * L, L)] for k in range(K)]
    beffv = beffvm[...]
    lane = lax.iota(jnp.int32, L)

    def do_rows(xin, yout, effb, nrows):
        @plsc.parallel_loop(0, nrows // L)
        def group_body(g):
            base_r = g * L
            z16 = jnp.zeros((L,), jnp.float32)
            for i in range(L):
                r = base_r + i
                ps = [xin[r, pl.ds(k * L, L)] * vv[k] for k in range(K)]
                while len(ps) > 1:
                    ps = [ps[j] + ps[j + 1] for j in range(0, len(ps), 2)]
                zs = jnp.sum(ps[0])
                z16 = jnp.where(lane == i, zs, z16)
            eff16 = 1.0 / (1.0 + jnp.exp(-(z16 + beffv)))
            effb[pl.ds(base_r, L)] = eff16
            for i in range(L):
                r = base_r + i
                e = eff16[i]
                for k in range(K):
                    yout[r, pl.ds(k * L, L)] = e * (
                        uu[k] * xin[r, pl.ds(k * L, L)] + bb[k])

    nchunks = (NSC - 1 - wid) // NW + 1

    bufs = ((xin0, yout0, effb0, isem0, osem0),
            (xin1, yout1, effb1, isem1, osem1))

    pltpu.make_async_copy(x_hbm.at[pl.ds((C0 + wid) * R, R)], xin0, isem0).start()

    def chunk_body(t, carry):
        base = (C0 + wid + t * NW) * R

        def phase(cur, nxt):
            xin, yout, effb, isem, osem = cur
            nxin, _, _, nisem, _ = nxt
            pltpu.make_async_copy(x_hbm.at[pl.ds(base, R)], xin, isem).wait()

            @pl.when(t + 1 < nchunks)
            def _prefetch():
                nbase = (C0 + wid + (t + 1) * NW) * R
                pltpu.make_async_copy(
                    x_hbm.at[pl.ds(nbase, R)], nxin, nisem).start()

            @pl.when(t >= 2)
            def _drain():
                pltpu.make_async_copy(
                    yout, y_hbm.at[pl.ds(base, R)], osem).wait()
                pltpu.make_async_copy(
                    effb, eff_hbm.at[pl.ds(base, R)], osem).wait()

            do_rows(xin, yout, effb, R)
            pltpu.make_async_copy(yout, y_hbm.at[pl.ds(base, R)], osem).start()
            pltpu.make_async_copy(effb, eff_hbm.at[pl.ds(base, R)], osem).start()

        @pl.when(t % 2 == 0)
        def _even():
            phase(bufs[0], bufs[1])

        @pl.when(t % 2 == 1)
        def _odd():
            phase(bufs[1], bufs[0])

        return carry

    lax.fori_loop(0, nchunks, chunk_body, 0)

    for (_, yout, effb, _, osem) in bufs:
        pltpu.make_async_copy(yout, y_hbm.at[pl.ds(0, R)], osem).wait()
        pltpu.make_async_copy(effb, eff_hbm.at[pl.ds(0, R)], osem).wait()

    @pl.when(wid == NW - 1)
    def _tail():
        pltpu.sync_copy(x_hbm.at[pl.ds(TAIL_BASE, TAIL)], xin0.at[pl.ds(0, TAIL)])
        do_rows(xin0, yout0, effb0, TAIL)
        pltpu.sync_copy(yout0.at[pl.ds(0, TAIL)],
                        y_hbm.at[pl.ds(TAIL_BASE, TAIL)])
        pltpu.sync_copy(effb0.at[pl.ds(0, TAIL)],
                        eff_hbm.at[pl.ds(TAIL_BASE, TAIL)])


@jax.jit
def _run(x, v, u, b, beff16):
    mesh = plsc.VectorSubcoreMesh(core_axis_name="c", subcore_axis_name="s",
                                  num_cores=NC, num_subcores=NS)
    eff, y = pl.kernel(
        _body,
        out_type=(jax.ShapeDtypeStruct((N,), jnp.float32),
                  jax.ShapeDtypeStruct((N, DA), jnp.float32)),
        mesh=mesh,
        compiler_params=pltpu.CompilerParams(needs_layout_passes=False),
        scratch_types=(
            pltpu.VMEM((R, DA), jnp.float32),
            pltpu.VMEM((R, DA), jnp.float32),
            pltpu.VMEM((R, DA), jnp.float32),
            pltpu.VMEM((R, DA), jnp.float32),
            pltpu.VMEM((R,), jnp.float32),
            pltpu.VMEM((R,), jnp.float32),
            pltpu.VMEM((DA,), jnp.float32),
            pltpu.VMEM((DA,), jnp.float32),
            pltpu.VMEM((DA,), jnp.float32),
            pltpu.VMEM((L,), jnp.float32),
            pltpu.SemaphoreType.DMA,
            pltpu.SemaphoreType.DMA,
            pltpu.SemaphoreType.DMA,
            pltpu.SemaphoreType.DMA,
        ),
    )(x, v, u, b, beff16)
    return eff, y


def _tc_body(beff_ref, v_ref, u_ref, b_ref, x_ref, ysc_ref, effsc_ref,
             y_ref, eff_ref):
    del ysc_ref, effsc_ref
    xb = x_ref[...]
    z = jnp.sum(xb * v_ref[...], axis=1, keepdims=True)
    eff = 1.0 / (1.0 + jnp.exp(-(z + beff_ref[0, 0])))
    eff_ref[...] = eff
    y_ref[...] = eff * (u_ref[...] * xb + b_ref[...])


@jax.jit
def _tc_run(beff, v2, u2, b2, x, y_sc, eff_sc):
    return pl.pallas_call(
        _tc_body,
        grid=(NT // BT,),
        in_specs=[
            pl.BlockSpec((1, 1), lambda i: (0, 0), memory_space=pltpu.SMEM),
            pl.BlockSpec((1, DA), lambda i: (0, 0)),
            pl.BlockSpec((1, DA), lambda i: (0, 0)),
            pl.BlockSpec((1, DA), lambda i: (0, 0)),
            pl.BlockSpec((BT, DA), lambda i: (i, 0)),
            pl.BlockSpec(memory_space=pl.ANY),
            pl.BlockSpec(memory_space=pl.ANY),
        ],
        out_specs=[
            pl.BlockSpec((BT, DA), lambda i: (i, 0)),
            pl.BlockSpec((BT, 1), lambda i: (i, 0)),
        ],
        out_shape=[
            jax.ShapeDtypeStruct((N, DA), jnp.float32),
            jax.ShapeDtypeStruct((N, 1), jnp.float32),
        ],
        input_output_aliases={5: 0, 6: 1},
    )(beff, v2, u2, b2, x, y_sc, eff_sc)


def kernel(x, n, W_eff, b_eff, w_t, b_t):
    v = n * W_eff[0]
    u = w_t * n
    beff16 = jnp.broadcast_to(b_eff[0], (L,))
    eff_sc, y_sc = _run(x, v, u, b_t, beff16)
    y, eff = _tc_run(b_eff.reshape(1, 1), v.reshape(1, DA), u.reshape(1, DA),
                     b_t.reshape(1, DA), x, y_sc, eff_sc.reshape(N, 1))
    return (eff, y)


# speedup vs baseline: 1.5016x; 1.5016x over previous
import jax
import jax.numpy as jnp
from jax import lax
from jax.experimental import pallas as pl
from jax.experimental.pallas import tpu as pltpu
from jax.experimental.pallas import tpu_sc as plsc

N = 100000
DA = 128
L = 16
NC = 2
NS = 16
NW = NC * NS
R = 128
NFULL = N // R
TAIL = N - NFULL * R
TAIL_BASE = NFULL * R
K = DA // L

BT = 1024
NT = 63 * BT
C0 = NT // R
NSC = NFULL - C0


def _body(x_hbm, v_hbm, u_hbm, b_hbm, beff_hbm, eff_hbm, y_hbm,
          xin0, xin1, yout0, yout1, effb0, effb1,
          vvm, uvm, bvm, beffvm, isem0, isem1, osem0, osem1):
    wid = lax.axis_index("s") * NC + lax.axis_index("c")

    pltpu.sync_copy(v_hbm, vvm)
    pltpu.sync_copy(u_hbm, uvm)
    pltpu.sync_copy(b_hbm, bvm)
    pltpu.sync_copy(beff_hbm, beffvm)

    vv = [vvm[pl.ds(k * L, L)] for k in range(K)]
    uu = [uvm[pl.ds(k * L, L)] for k in range(K)]
    bb = [bvm[pl.ds(k * L, L)] for k in range(K)]
    beffv = beffvm[...]
    lane = lax.iota(jnp.int32, L)

    def do_rows(xin, yout, effb, nrows):
        @plsc.parallel_loop(0, nrows // L)
        def group_body(g):
            base_r = g * L
            z16 = jnp.zeros((L,), jnp.float32)
            for i in range(L):
                r = base_r + i
                ps = [xin[r, pl.ds(k * L, L)] * vv[k] for k in range(K)]
                while len(ps) > 1:
                    ps = [ps[j] + ps[j + 1] for j in range(0, len(ps), 2)]
                zs = jnp.sum(ps[0])
                z16 = jnp.where(lane == i, zs, z16)
            eff16 = 1.0 / (1.0 + jnp.exp(-(z16 + beffv)))
            effb[pl.ds(base_r, L)] = eff16
            for i in range(L):
                r = base_r + i
                e = eff16[i]
                for k in range(K):
                    yout[r, pl.ds(k * L, L)] = e * (
                        uu[k] * xin[r, pl.ds(k * L, L)] + bb[k])

    nchunks = (NSC - 1 - wid) // NW + 1

    bufs = ((xin0, yout0, effb0, isem0, osem0),
            (xin1, yout1, effb1, isem1, osem1))

    pltpu.make_async_copy(x_hbm.at[pl.ds((C0 + wid) * R, R)], xin0, isem0).start()

    def chunk_body(t, carry):
        base = (C0 + wid + t * NW) * R

        def phase(cur, nxt):
            xin, yout, effb, isem, osem = cur
            nxin, _, _, nisem, _ = nxt
            pltpu.make_async_copy(x_hbm.at[pl.ds(base, R)], xin, isem).wait()

            @pl.when(t + 1 < nchunks)
            def _prefetch():
                nbase = (C0 + wid + (t + 1) * NW) * R
                pltpu.make_async_copy(
                    x_hbm.at[pl.ds(nbase, R)], nxin, nisem).start()

            @pl.when(t >= 2)
            def _drain():
                pltpu.make_async_copy(
                    yout, y_hbm.at[pl.ds(base, R)], osem).wait()
                pltpu.make_async_copy(
                    effb, eff_hbm.at[pl.ds(base, R)], osem).wait()

            do_rows(xin, yout, effb, R)
            pltpu.make_async_copy(yout, y_hbm.at[pl.ds(base, R)], osem).start()
            pltpu.make_async_copy(effb, eff_hbm.at[pl.ds(base, R)], osem).start()

        @pl.when(t % 2 == 0)
        def _even():
            phase(bufs[0], bufs[1])

        @pl.when(t % 2 == 1)
        def _odd():
            phase(bufs[1], bufs[0])

        return carry

    lax.fori_loop(0, nchunks, chunk_body, 0)

    for (_, yout, effb, _, osem) in bufs:
        pltpu.make_async_copy(yout, y_hbm.at[pl.ds(0, R)], osem).wait()
        pltpu.make_async_copy(effb, eff_hbm.at[pl.ds(0, R)], osem).wait()

    @pl.when(wid == NW - 1)
    def _tail():
        pltpu.sync_copy(x_hbm.at[pl.ds(TAIL_BASE, TAIL)], xin0.at[pl.ds(0, TAIL)])
        do_rows(xin0, yout0, effb0, TAIL)
        pltpu.sync_copy(yout0.at[pl.ds(0, TAIL)],
                        y_hbm.at[pl.ds(TAIL_BASE, TAIL)])
        pltpu.sync_copy(effb0.at[pl.ds(0, TAIL)],
                        eff_hbm.at[pl.ds(TAIL_BASE, TAIL)])


@jax.jit
def _run(x, v, u, b, beff16):
    mesh = plsc.VectorSubcoreMesh(core_axis_name="c", subcore_axis_name="s",
                                  num_cores=NC, num_subcores=NS)
    eff, y = pl.kernel(
        _body,
        out_type=(jax.ShapeDtypeStruct((N,), jnp.float32),
                  jax.ShapeDtypeStruct((N, DA), jnp.float32)),
        mesh=mesh,
        compiler_params=pltpu.CompilerParams(needs_layout_passes=False),
        scratch_types=(
            pltpu.VMEM((R, DA), jnp.float32),
            pltpu.VMEM((R, DA), jnp.float32),
            pltpu.VMEM((R, DA), jnp.float32),
            pltpu.VMEM((R, DA), jnp.float32),
            pltpu.VMEM((R,), jnp.float32),
            pltpu.VMEM((R,), jnp.float32),
            pltpu.VMEM((DA,), jnp.float32),
            pltpu.VMEM((DA,), jnp.float32),
            pltpu.VMEM((DA,), jnp.float32),
            pltpu.VMEM((L,), jnp.float32),
            pltpu.SemaphoreType.DMA,
            pltpu.SemaphoreType.DMA,
            pltpu.SemaphoreType.DMA,
            pltpu.SemaphoreType.DMA,
        ),
    )(x, v, u, b, beff16)
    return eff, y


def _tc_body(beff_ref, v_ref, u_ref, b_ref, x_ref, ysc_ref,
             y_ref, eff_ref):
    del ysc_ref
    xb = x_ref[...]
    z = jnp.sum(xb * v_ref[...], axis=1, keepdims=True)
    eff = 1.0 / (1.0 + jnp.exp(-(z + beff_ref[0, 0])))
    eff_ref[...] = eff[:, 0]
    y_ref[...] = eff * (u_ref[...] * xb + b_ref[...])


@jax.jit
def _tc_run(beff, v2, u2, b2, x, y_sc):
    return pl.pallas_call(
        _tc_body,
        grid=(NT // BT,),
        in_specs=[
            pl.BlockSpec((1, 1), lambda i: (0, 0), memory_space=pltpu.SMEM),
            pl.BlockSpec((1, DA), lambda i: (0, 0)),
            pl.BlockSpec((1, DA), lambda i: (0, 0)),
            pl.BlockSpec((1, DA), lambda i: (0, 0)),
            pl.BlockSpec((BT, DA), lambda i: (i, 0)),
            pl.BlockSpec(memory_space=pl.ANY),
        ],
        out_specs=[
            pl.BlockSpec((BT, DA), lambda i: (i, 0)),
            pl.BlockSpec((BT,), lambda i: (i,)),
        ],
        out_shape=[
            jax.ShapeDtypeStruct((N, DA), jnp.float32),
            jax.ShapeDtypeStruct((NT,), jnp.float32),
        ],
        input_output_aliases={5: 0},
    )(beff, v2, u2, b2, x, y_sc)


def kernel(x, n, W_eff, b_eff, w_t, b_t):
    v = n * W_eff[0]
    u = w_t * n
    beff16 = jnp.broadcast_to(b_eff[0], (L,))
    eff_sc, y_sc = _run(x, v, u, b_t, beff16)
    y, eff_tc = _tc_run(b_eff.reshape(1, 1), v.reshape(1, DA), u.reshape(1, DA),
                        b_t.reshape(1, DA), x, y_sc)
    eff = jnp.concatenate([eff_tc, eff_sc[NT:]])
    return (eff.reshape(N, 1), y)


# speedup vs baseline: 1.6880x; 1.1241x over previous
import jax
import jax.numpy as jnp
from jax import lax
from jax.experimental import pallas as pl
from jax.experimental.pallas import tpu as pltpu
from jax.experimental.pallas import tpu_sc as plsc

N = 100000
DA = 128
L = 16
NC = 2
NS = 16
NW = NC * NS
R = 128
NFULL = N // R
TAIL = N - NFULL * R
TAIL_BASE = NFULL * R
K = DA // L

BT = 2048
NT = 31 * BT
C0 = NT // R
NSC = NFULL - C0


def _body(x_hbm, v_hbm, u_hbm, b_hbm, beff_hbm, eff_hbm, y_hbm,
          xin0, xin1, yout0, yout1, effb0, effb1,
          vvm, uvm, bvm, beffvm, isem0, isem1, osem0, osem1):
    wid = lax.axis_index("s") * NC + lax.axis_index("c")

    pltpu.sync_copy(v_hbm, vvm)
    pltpu.sync_copy(u_hbm, uvm)
    pltpu.sync_copy(b_hbm, bvm)
    pltpu.sync_copy(beff_hbm, beffvm)

    vv = [vvm[pl.ds(k * L, L)] for k in range(K)]
    uu = [uvm[pl.ds(k * L, L)] for k in range(K)]
    bb = [bvm[pl.ds(k * L, L)] for k in range(K)]
    beffv = beffvm[...]
    lane = lax.iota(jnp.int32, L)

    def do_rows(xin, yout, effb, nrows):
        @plsc.parallel_loop(0, nrows // L)
        def group_body(g):
            base_r = g * L
            z16 = jnp.zeros((L,), jnp.float32)
            for i in range(L):
                r = base_r + i
                ps = [xin[r, pl.ds(k * L, L)] * vv[k] for k in range(K)]
                while len(ps) > 1:
                    ps = [ps[j] + ps[j + 1] for j in range(0, len(ps), 2)]
                zs = jnp.sum(ps[0])
                z16 = jnp.where(lane == i, zs, z16)
            eff16 = 1.0 / (1.0 + jnp.exp(-(z16 + beffv)))
            effb[pl.ds(base_r, L)] = eff16
            for i in range(L):
                r = base_r + i
                e = eff16[i]
                for k in range(K):
                    yout[r, pl.ds(k * L, L)] = e * (
                        uu[k] * xin[r, pl.ds(k * L, L)] + bb[k])

    nchunks = (NSC - 1 - wid) // NW + 1

    bufs = ((xin0, yout0, effb0, isem0, osem0),
            (xin1, yout1, effb1, isem1, osem1))

    pltpu.make_async_copy(x_hbm.at[pl.ds((C0 + wid) * R, R)], xin0, isem0).start()

    def chunk_body(t, carry):
        base = (C0 + wid + t * NW) * R

        def phase(cur, nxt):
            xin, yout, effb, isem, osem = cur
            nxin, _, _, nisem, _ = nxt
            pltpu.make_async_copy(x_hbm.at[pl.ds(base, R)], xin, isem).wait()

            @pl.when(t + 1 < nchunks)
            def _prefetch():
                nbase = (C0 + wid + (t + 1) * NW) * R
                pltpu.make_async_copy(
                    x_hbm.at[pl.ds(nbase, R)], nxin, nisem).start()

            @pl.when(t >= 2)
            def _drain():
                pltpu.make_async_copy(
                    yout, y_hbm.at[pl.ds(base, R)], osem).wait()
                pltpu.make_async_copy(
                    effb, eff_hbm.at[pl.ds(base, R)], osem).wait()

            do_rows(xin, yout, effb, R)
            pltpu.make_async_copy(yout, y_hbm.at[pl.ds(base, R)], osem).start()
            pltpu.make_async_copy(effb, eff_hbm.at[pl.ds(base, R)], osem).start()

        @pl.when(t % 2 == 0)
        def _even():
            phase(bufs[0], bufs[1])

        @pl.when(t % 2 == 1)
        def _odd():
            phase(bufs[1], bufs[0])

        return carry

    lax.fori_loop(0, nchunks, chunk_body, 0)

    for (_, yout, effb, _, osem) in bufs:
        pltpu.make_async_copy(yout, y_hbm.at[pl.ds(0, R)], osem).wait()
        pltpu.make_async_copy(effb, eff_hbm.at[pl.ds(0, R)], osem).wait()

    @pl.when(wid == NW - 1)
    def _tail():
        pltpu.sync_copy(x_hbm.at[pl.ds(TAIL_BASE, TAIL)], xin0.at[pl.ds(0, TAIL)])
        do_rows(xin0, yout0, effb0, TAIL)
        pltpu.sync_copy(yout0.at[pl.ds(0, TAIL)],
                        y_hbm.at[pl.ds(TAIL_BASE, TAIL)])
        pltpu.sync_copy(effb0.at[pl.ds(0, TAIL)],
                        eff_hbm.at[pl.ds(TAIL_BASE, TAIL)])


@jax.jit
def _run(x, v, u, b, beff16):
    mesh = plsc.VectorSubcoreMesh(core_axis_name="c", subcore_axis_name="s",
                                  num_cores=NC, num_subcores=NS)
    eff, y = pl.kernel(
        _body,
        out_type=(jax.ShapeDtypeStruct((N,), jnp.float32),
                  jax.ShapeDtypeStruct((N, DA), jnp.float32)),
        mesh=mesh,
        compiler_params=pltpu.CompilerParams(needs_layout_passes=False),
        scratch_types=(
            pltpu.VMEM((R, DA), jnp.float32),
            pltpu.VMEM((R, DA), jnp.float32),
            pltpu.VMEM((R, DA), jnp.float32),
            pltpu.VMEM((R, DA), jnp.float32),
            pltpu.VMEM((R,), jnp.float32),
            pltpu.VMEM((R,), jnp.float32),
            pltpu.VMEM((DA,), jnp.float32),
            pltpu.VMEM((DA,), jnp.float32),
            pltpu.VMEM((DA,), jnp.float32),
            pltpu.VMEM((L,), jnp.float32),
            pltpu.SemaphoreType.DMA,
            pltpu.SemaphoreType.DMA,
            pltpu.SemaphoreType.DMA,
            pltpu.SemaphoreType.DMA,
        ),
    )(x, v, u, b, beff16)
    return eff, y


def _tc_body(beff_ref, v_ref, u_ref, b_ref, x_ref, ysc_ref,
             y_ref, eff_ref):
    del ysc_ref
    xb = x_ref[...]
    z = jax.lax.dot(xb, v_ref[...])
    eff = 1.0 / (1.0 + jnp.exp(-(z + beff_ref[0, 0])))
    eff_ref[...] = eff[:, 0]
    y_ref[...] = eff * (u_ref[...] * xb + b_ref[...])


@jax.jit
def _tc_run(beff, v2, u2, b2, x, y_sc):
    return pl.pallas_call(
        _tc_body,
        grid=(NT // BT,),
        in_specs=[
            pl.BlockSpec((1, 1), lambda i: (0, 0), memory_space=pltpu.SMEM),
            pl.BlockSpec((DA, 1), lambda i: (0, 0)),
            pl.BlockSpec((1, DA), lambda i: (0, 0)),
            pl.BlockSpec((1, DA), lambda i: (0, 0)),
            pl.BlockSpec((BT, DA), lambda i: (i, 0)),
            pl.BlockSpec(memory_space=pl.ANY),
        ],
        out_specs=[
            pl.BlockSpec((BT, DA), lambda i: (i, 0)),
            pl.BlockSpec((BT,), lambda i: (i,)),
        ],
        out_shape=[
            jax.ShapeDtypeStruct((N, DA), jnp.float32),
            jax.ShapeDtypeStruct((NT,), jnp.float32),
        ],
        input_output_aliases={5: 0},
    )(beff, v2, u2, b2, x, y_sc)


def kernel(x, n, W_eff, b_eff, w_t, b_t):
    v = n * W_eff[0]
    u = w_t * n
    beff16 = jnp.broadcast_to(b_eff[0], (L,))
    eff_sc, y_sc = _run(x, v, u, b_t, beff16)
    y, eff_tc = _tc_run(b_eff.reshape(1, 1), v.reshape(DA, 1), u.reshape(1, DA),
                        b_t.reshape(1, DA), x, y_sc)
    eff = jnp.concatenate([eff_tc, eff_sc[NT:]])
    return (eff.reshape(N, 1), y)


# speedup vs baseline: 1.7681x; 1.0475x over previous
import jax
import jax.numpy as jnp
from jax import lax
from jax.experimental import pallas as pl
from jax.experimental.pallas import tpu as pltpu
from jax.experimental.pallas import tpu_sc as plsc

N = 100000
DA = 128
L = 16
NC = 2
NS = 16
NW = NC * NS
R = 128
NFULL = N // R
TAIL = N - NFULL * R
TAIL_BASE = NFULL * R
K = DA // L

BT = 8192
NT = 8 * BT
C0 = NT // R
NSC = NFULL - C0


def _body(x_hbm, v_hbm, u_hbm, b_hbm, beff_hbm, eff_hbm, y_hbm,
          xin0, xin1, yout0, yout1, effb0, effb1,
          vvm, uvm, bvm, beffvm, isem0, isem1, osem0, osem1):
    wid = lax.axis_index("s") * NC + lax.axis_index("c")

    pltpu.sync_copy(v_hbm, vvm)
    pltpu.sync_copy(u_hbm, uvm)
    pltpu.sync_copy(b_hbm, bvm)
    pltpu.sync_copy(beff_hbm, beffvm)

    vv = [vvm[pl.ds(k * L, L)] for k in range(K)]
    uu = [uvm[pl.ds(k * L, L)] for k in range(K)]
    bb = [bvm[pl.ds(k * L, L)] for k in range(K)]
    beffv = beffvm[...]
    lane = lax.iota(jnp.int32, L)

    def do_rows(xin, yout, effb, nrows):
        @plsc.parallel_loop(0, nrows // L)
        def group_body(g):
            base_r = g * L
            z16 = jnp.zeros((L,), jnp.float32)
            for i in range(L):
                r = base_r + i
                ps = [xin[r, pl.ds(k * L, L)] * vv[k] for k in range(K)]
                while len(ps) > 1:
                    ps = [ps[j] + ps[j + 1] for j in range(0, len(ps), 2)]
                zs = jnp.sum(ps[0])
                z16 = jnp.where(lane == i, zs, z16)
            eff16 = 1.0 / (1.0 + jnp.exp(-(z16 + beffv)))
            effb[pl.ds(base_r, L)] = eff16
            for i in range(L):
                r = base_r + i
                e = eff16[i]
                for k in range(K):
                    yout[r, pl.ds(k * L, L)] = e * (
                        uu[k] * xin[r, pl.ds(k * L, L)] + bb[k])

    nchunks = (NSC - 1 - wid) // NW + 1

    bufs = ((xin0, yout0, effb0, isem0, osem0),
            (xin1, yout1, effb1, isem1, osem1))

    pltpu.make_async_copy(x_hbm.at[pl.ds((C0 + wid) * R, R)], xin0, isem0).start()

    def chunk_body(t, carry):
        base = (C0 + wid + t * NW) * R

        def phase(cur, nxt):
            xin, yout, effb, isem, osem = cur
            nxin, _, _, nisem, _ = nxt
            pltpu.make_async_copy(x_hbm.at[pl.ds(base, R)], xin, isem).wait()

            @pl.when(t + 1 < nchunks)
            def _prefetch():
                nbase = (C0 + wid + (t + 1) * NW) * R
                pltpu.make_async_copy(
                    x_hbm.at[pl.ds(nbase, R)], nxin, nisem).start()

            @pl.when(t >= 2)
            def _drain():
                pltpu.make_async_copy(
                    yout, y_hbm.at[pl.ds(base, R)], osem).wait()
                pltpu.make_async_copy(
                    effb, eff_hbm.at[pl.ds(base, R)], osem).wait()

            do_rows(xin, yout, effb, R)
            pltpu.make_async_copy(yout, y_hbm.at[pl.ds(base, R)], osem).start()
            pltpu.make_async_copy(effb, eff_hbm.at[pl.ds(base, R)], osem).start()

        @pl.when(t % 2 == 0)
        def _even():
            phase(bufs[0], bufs[1])

        @pl.when(t % 2 == 1)
        def _odd():
            phase(bufs[1], bufs[0])

        return carry

    lax.fori_loop(0, nchunks, chunk_body, 0)

    for (_, yout, effb, _, osem) in bufs:
        pltpu.make_async_copy(yout, y_hbm.at[pl.ds(0, R)], osem).wait()
        pltpu.make_async_copy(effb, eff_hbm.at[pl.ds(0, R)], osem).wait()

    @pl.when(wid == NW - 1)
    def _tail():
        pltpu.sync_copy(x_hbm.at[pl.ds(TAIL_BASE, TAIL)], xin0.at[pl.ds(0, TAIL)])
        do_rows(xin0, yout0, effb0, TAIL)
        pltpu.sync_copy(yout0.at[pl.ds(0, TAIL)],
                        y_hbm.at[pl.ds(TAIL_BASE, TAIL)])
        pltpu.sync_copy(effb0.at[pl.ds(0, TAIL)],
                        eff_hbm.at[pl.ds(TAIL_BASE, TAIL)])


@jax.jit
def _run(x, v, u, b, beff16):
    mesh = plsc.VectorSubcoreMesh(core_axis_name="c", subcore_axis_name="s",
                                  num_cores=NC, num_subcores=NS)
    eff, y = pl.kernel(
        _body,
        out_type=(jax.ShapeDtypeStruct((N,), jnp.float32),
                  jax.ShapeDtypeStruct((N, DA), jnp.float32)),
        mesh=mesh,
        compiler_params=pltpu.CompilerParams(needs_layout_passes=False),
        scratch_types=(
            pltpu.VMEM((R, DA), jnp.float32),
            pltpu.VMEM((R, DA), jnp.float32),
            pltpu.VMEM((R, DA), jnp.float32),
            pltpu.VMEM((R, DA), jnp.float32),
            pltpu.VMEM((R,), jnp.float32),
            pltpu.VMEM((R,), jnp.float32),
            pltpu.VMEM((DA,), jnp.float32),
            pltpu.VMEM((DA,), jnp.float32),
            pltpu.VMEM((DA,), jnp.float32),
            pltpu.VMEM((L,), jnp.float32),
            pltpu.SemaphoreType.DMA,
            pltpu.SemaphoreType.DMA,
            pltpu.SemaphoreType.DMA,
            pltpu.SemaphoreType.DMA,
        ),
    )(x, v, u, b, beff16)
    return eff, y


def _tc_body(beff_ref, v_ref, u_ref, b_ref, x_ref, ysc_ref,
             y_ref, eff_ref):
    del ysc_ref
    xb = x_ref[...]
    z = jax.lax.dot(xb, v_ref[...])
    eff = 1.0 / (1.0 + jnp.exp(-(z + beff_ref[0, 0])))
    eff_ref[...] = eff[:, 0]
    y_ref[...] = eff * (u_ref[...] * xb + b_ref[...])


@jax.jit
def _tc_run(beff, v2, u2, b2, x, y_sc):
    return pl.pallas_call(
        _tc_body,
        grid=(NT // BT,),
        in_specs=[
            pl.BlockSpec((1, 1), lambda i: (0, 0), memory_space=pltpu.SMEM),
            pl.BlockSpec((DA, 1), lambda i: (0, 0)),
            pl.BlockSpec((1, DA), lambda i: (0, 0)),
            pl.BlockSpec((1, DA), lambda i: (0, 0)),
            pl.BlockSpec((BT, DA), lambda i: (i, 0)),
            pl.BlockSpec(memory_space=pl.ANY),
        ],
        out_specs=[
            pl.BlockSpec((BT, DA), lambda i: (i, 0)),
            pl.BlockSpec((BT,), lambda i: (i,)),
        ],
        out_shape=[
            jax.ShapeDtypeStruct((N, DA), jnp.float32),
            jax.ShapeDtypeStruct((NT,), jnp.float32),
        ],
        input_output_aliases={5: 0},
    )(beff, v2, u2, b2, x, y_sc)


def kernel(x, n, W_eff, b_eff, w_t, b_t):
    v = n * W_eff[0]
    u = w_t * n
    beff16 = jnp.broadcast_to(b_eff[0], (L,))
    eff_sc, y_sc = _run(x, v, u, b_t, beff16)
    y, eff_tc = _tc_run(b_eff.reshape(1, 1), v.reshape(DA, 1), u.reshape(1, DA),
                        b_t.reshape(1, DA), x, y_sc)
    eff = jnp.concatenate([eff_tc, eff_sc[NT:]])
    return (eff.reshape(N, 1), y)
